# Initial kernel scaffold; baseline (speedup 1.0000x reference)
import jax
import jax.numpy as jnp
from jax.experimental import pallas as pl


def kernel(x, Wr, W1, W2, gamma):
    raise NotImplementedError("write your pallas kernel here")


# speedup vs baseline: 1.3092x; 1.3092x over previous
import jax
import jax.numpy as jnp
from jax.experimental import pallas as pl
from jax.experimental.pallas import tpu as pltpu

_B, _S, _D = 4, 8192, 1024
_DFF = 2048
_R = 4
_T = _B * _S
_TILE = 512


def _mor_kernel(x_ref, wr_ref, w1_ref, w2_ref, g_ref, out_ref, probs_ref):
    xt = x_ref[...]
    logits = jnp.dot(xt, wr_ref[...], preferred_element_type=jnp.float32)
    m = jnp.max(logits, axis=-1, keepdims=True)
    e = jnp.exp(logits - m)
    probs = e / jnp.sum(e, axis=-1, keepdims=True)
    probs_ref[...] = probs
    top_idx = jnp.argmax(probs, axis=-1)[:, None]
    top_w = jnp.max(probs, axis=-1, keepdims=True)

    h = xt
    final = xt
    for r in range(_R):
        var = jnp.mean(h * h, axis=-1, keepdims=True)
        hn = h * jax.lax.rsqrt(var + 1e-6) * g_ref[r]
        up = jnp.dot(hn.astype(jnp.bfloat16), w1_ref[r],
                     preferred_element_type=jnp.float32)
        act = up * jax.nn.sigmoid(up)
        out = jnp.dot(act.astype(jnp.bfloat16), w2_ref[r],
                      preferred_element_type=jnp.float32)
        h = jnp.where(top_idx >= r, h + out, h)
        final = jnp.where(top_idx == r, h * top_w + xt, final)
    out_ref[...] = final


def kernel(x, Wr, W1, W2, gamma):
    x2 = x.reshape(_T, _D)
    w1 = W1.astype(jnp.bfloat16)
    w2 = W2.astype(jnp.bfloat16)
    grid = (_T // _TILE,)
    out, probs = pl.pallas_call(
        _mor_kernel,
        grid=grid,
        in_specs=[
            pl.BlockSpec((_TILE, _D), lambda i: (i, 0)),
            pl.BlockSpec((_D, _R), lambda i: (0, 0)),
            pl.BlockSpec((_R, _D, _DFF), lambda i: (0, 0, 0)),
            pl.BlockSpec((_R, _DFF, _D), lambda i: (0, 0, 0)),
            pl.BlockSpec((_R, _D), lambda i: (0, 0)),
        ],
        out_specs=[
            pl.BlockSpec((_TILE, _D), lambda i: (i, 0)),
            pl.BlockSpec((_TILE, _R), lambda i: (i, 0)),
        ],
        out_shape=[
            jax.ShapeDtypeStruct((_T, _D), jnp.float32),
            jax.ShapeDtypeStruct((_T, _R), jnp.float32),
        ],
    )(x2, Wr, w1, w2, gamma)
    return out.reshape(_B, _S, _D), probs.reshape(_B, _S, _R)
